# baseline (device time: 31853 ns/iter reference)
import jax
import jax.numpy as jnp
from jax import lax
from jax.experimental import pallas as pl
from jax.experimental.pallas import tpu as pltpu

NB = 8


def kernel(x):
    m_per, n = x.shape
    m_q = m_per // 4
    blk = m_q // NB

    def body(x_ref, out_ref, copy_sem,
             y_send, y_recv, x2_send, x2_recv, z2_send, z2_recv,
             x3_send, x3_recv, z3_send, z3_recv):
        my_x = lax.axis_index("x")
        my_y = lax.axis_index("y")
        my_z = lax.axis_index("z")
        y_peer = (my_x, 1 - my_y, my_z)
        x_peer = (1 - my_x, my_y, my_z)
        z_peer = (my_x, my_y, 1 - my_z)

        barrier_sem = pltpu.get_barrier_semaphore()
        for peer in (y_peer, x_peer, z_peer):
            pl.semaphore_signal(
                barrier_sem, inc=1,
                device_id=peer, device_id_type=pl.DeviceIdType.MESH,
            )
        pl.semaphore_wait(barrier_sem, 3)

        own_base = my_y * m_per
        recv_base = (1 - my_y) * m_per

        q_mine = (2 * my_x + my_z) * m_q
        q_x = (2 * (1 - my_x) + my_z) * m_q
        q_z = (2 * my_x + (1 - my_z)) * m_q
        q_diag = (2 * (1 - my_x) + (1 - my_z)) * m_q

        local_copy = pltpu.make_async_copy(
            x_ref, out_ref.at[pl.ds(own_base, m_per), :], copy_sem,
        )
        local_copy.start()

        y_rdmas = []
        for i in range(NB):
            r = pltpu.make_async_remote_copy(
                src_ref=x_ref.at[pl.ds(q_mine + i * blk, blk), :],
                dst_ref=out_ref.at[pl.ds(own_base + q_mine + i * blk, blk), :],
                send_sem=y_send.at[i],
                recv_sem=y_recv.at[i],
                device_id=y_peer,
                device_id_type=pl.DeviceIdType.MESH,
            )
            r.start()
            y_rdmas.append(r)

        x2_rdmas, z2_rdmas = [], []
        for i in range(NB):
            y_rdmas[i].wait_recv()
            src = out_ref.at[pl.ds(recv_base + q_mine + i * blk, blk), :]
            for peer, sends, recvs, lst in (
                (x_peer, x2_send, x2_recv, x2_rdmas),
                (z_peer, z2_send, z2_recv, z2_rdmas),
            ):
                r = pltpu.make_async_remote_copy(
                    src_ref=src,
                    dst_ref=out_ref.at[pl.ds(recv_base + q_mine + i * blk, blk), :],
                    send_sem=sends.at[i],
                    recv_sem=recvs.at[i],
                    device_id=peer,
                    device_id_type=pl.DeviceIdType.MESH,
                )
                r.start()
                lst.append(r)

        half = NB // 2
        x3_rdmas, z3_rdmas = [], []
        for i in range(half):
            z2_rdmas[i].wait_recv()
            r = pltpu.make_async_remote_copy(
                src_ref=out_ref.at[pl.ds(recv_base + q_z + i * blk, blk), :],
                dst_ref=out_ref.at[pl.ds(recv_base + q_z + i * blk, blk), :],
                send_sem=x3_send.at[i],
                recv_sem=x3_recv.at[i],
                device_id=x_peer,
                device_id_type=pl.DeviceIdType.MESH,
            )
            r.start()
            x3_rdmas.append(r)
        for i in range(half, NB):
            x2_rdmas[i].wait_recv()
            j = i - half
            r = pltpu.make_async_remote_copy(
                src_ref=out_ref.at[pl.ds(recv_base + q_x + i * blk, blk), :],
                dst_ref=out_ref.at[pl.ds(recv_base + q_x + i * blk, blk), :],
                send_sem=z3_send.at[j],
                recv_sem=z3_recv.at[j],
                device_id=z_peer,
                device_id_type=pl.DeviceIdType.MESH,
            )
            r.start()
            z3_rdmas.append(r)
        for i in range(half, NB):
            z2_rdmas[i].wait_recv()
        for i in range(half):
            x2_rdmas[i].wait_recv()

        for j in range(half):
            x3_rdmas[j].wait_recv()
            z3_rdmas[j].wait_recv()

        for i in range(NB):
            y_rdmas[i].wait_send()
            x2_rdmas[i].wait_send()
            z2_rdmas[i].wait_send()
        for j in range(half):
            x3_rdmas[j].wait_send()
            z3_rdmas[j].wait_send()
        local_copy.wait()

    return pl.pallas_call(
        body,
        out_shape=jax.ShapeDtypeStruct((2 * m_per, n), x.dtype),
        in_specs=[pl.BlockSpec(memory_space=pl.ANY)],
        out_specs=pl.BlockSpec(memory_space=pl.ANY),
        scratch_shapes=[
            pltpu.SemaphoreType.DMA,
            pltpu.SemaphoreType.DMA((NB,)),
            pltpu.SemaphoreType.DMA((NB,)),
            pltpu.SemaphoreType.DMA((NB,)),
            pltpu.SemaphoreType.DMA((NB,)),
            pltpu.SemaphoreType.DMA((NB,)),
            pltpu.SemaphoreType.DMA((NB,)),
            pltpu.SemaphoreType.DMA((NB // 2,)),
            pltpu.SemaphoreType.DMA((NB // 2,)),
            pltpu.SemaphoreType.DMA((NB // 2,)),
            pltpu.SemaphoreType.DMA((NB // 2,)),
        ],
        compiler_params=pltpu.CompilerParams(collective_id=0),
    )(x)


# device time: 31763 ns/iter; 1.0028x vs baseline; 1.0028x over previous
import jax
import jax.numpy as jnp
from jax import lax
from jax.experimental import pallas as pl
from jax.experimental.pallas import tpu as pltpu

NB = 16


def kernel(x):
    m_per, n = x.shape
    m_q = m_per // 4
    blk = m_q // NB

    def body(x_ref, out_ref, copy_sem,
             y_send, y_recv, x2_send, x2_recv, z2_send, z2_recv,
             x3_send, x3_recv, z3_send, z3_recv):
        my_x = lax.axis_index("x")
        my_y = lax.axis_index("y")
        my_z = lax.axis_index("z")
        y_peer = (my_x, 1 - my_y, my_z)
        x_peer = (1 - my_x, my_y, my_z)
        z_peer = (my_x, my_y, 1 - my_z)

        barrier_sem = pltpu.get_barrier_semaphore()
        for peer in (y_peer, x_peer, z_peer):
            pl.semaphore_signal(
                barrier_sem, inc=1,
                device_id=peer, device_id_type=pl.DeviceIdType.MESH,
            )
        pl.semaphore_wait(barrier_sem, 3)

        own_base = my_y * m_per
        recv_base = (1 - my_y) * m_per

        q_mine = (2 * my_x + my_z) * m_q
        q_x = (2 * (1 - my_x) + my_z) * m_q
        q_z = (2 * my_x + (1 - my_z)) * m_q
        q_diag = (2 * (1 - my_x) + (1 - my_z)) * m_q

        local_copy = pltpu.make_async_copy(
            x_ref, out_ref.at[pl.ds(own_base, m_per), :], copy_sem,
        )
        local_copy.start()

        y_rdmas = []
        for i in range(NB):
            r = pltpu.make_async_remote_copy(
                src_ref=x_ref.at[pl.ds(q_mine + i * blk, blk), :],
                dst_ref=out_ref.at[pl.ds(own_base + q_mine + i * blk, blk), :],
                send_sem=y_send.at[i],
                recv_sem=y_recv.at[i],
                device_id=y_peer,
                device_id_type=pl.DeviceIdType.MESH,
            )
            r.start()
            y_rdmas.append(r)

        x2_rdmas, z2_rdmas = [], []
        for i in range(NB):
            y_rdmas[i].wait_recv()
            src = out_ref.at[pl.ds(recv_base + q_mine + i * blk, blk), :]
            for peer, sends, recvs, lst in (
                (x_peer, x2_send, x2_recv, x2_rdmas),
                (z_peer, z2_send, z2_recv, z2_rdmas),
            ):
                r = pltpu.make_async_remote_copy(
                    src_ref=src,
                    dst_ref=out_ref.at[pl.ds(recv_base + q_mine + i * blk, blk), :],
                    send_sem=sends.at[i],
                    recv_sem=recvs.at[i],
                    device_id=peer,
                    device_id_type=pl.DeviceIdType.MESH,
                )
                r.start()
                lst.append(r)

        half = NB // 2
        x3_rdmas, z3_rdmas = [], []
        for i in range(half):
            z2_rdmas[i].wait_recv()
            r = pltpu.make_async_remote_copy(
                src_ref=out_ref.at[pl.ds(recv_base + q_z + i * blk, blk), :],
                dst_ref=out_ref.at[pl.ds(recv_base + q_z + i * blk, blk), :],
                send_sem=x3_send.at[i],
                recv_sem=x3_recv.at[i],
                device_id=x_peer,
                device_id_type=pl.DeviceIdType.MESH,
            )
            r.start()
            x3_rdmas.append(r)
        for i in range(half, NB):
            x2_rdmas[i].wait_recv()
            j = i - half
            r = pltpu.make_async_remote_copy(
                src_ref=out_ref.at[pl.ds(recv_base + q_x + i * blk, blk), :],
                dst_ref=out_ref.at[pl.ds(recv_base + q_x + i * blk, blk), :],
                send_sem=z3_send.at[j],
                recv_sem=z3_recv.at[j],
                device_id=z_peer,
                device_id_type=pl.DeviceIdType.MESH,
            )
            r.start()
            z3_rdmas.append(r)
        for i in range(half, NB):
            z2_rdmas[i].wait_recv()
        for i in range(half):
            x2_rdmas[i].wait_recv()

        for j in range(half):
            x3_rdmas[j].wait_recv()
            z3_rdmas[j].wait_recv()

        for i in range(NB):
            y_rdmas[i].wait_send()
            x2_rdmas[i].wait_send()
            z2_rdmas[i].wait_send()
        for j in range(half):
            x3_rdmas[j].wait_send()
            z3_rdmas[j].wait_send()
        local_copy.wait()

    return pl.pallas_call(
        body,
        out_shape=jax.ShapeDtypeStruct((2 * m_per, n), x.dtype),
        in_specs=[pl.BlockSpec(memory_space=pltpu.VMEM)],
        out_specs=pl.BlockSpec(memory_space=pltpu.VMEM),
        scratch_shapes=[
            pltpu.SemaphoreType.DMA,
            pltpu.SemaphoreType.DMA((NB,)),
            pltpu.SemaphoreType.DMA((NB,)),
            pltpu.SemaphoreType.DMA((NB,)),
            pltpu.SemaphoreType.DMA((NB,)),
            pltpu.SemaphoreType.DMA((NB,)),
            pltpu.SemaphoreType.DMA((NB,)),
            pltpu.SemaphoreType.DMA((NB // 2,)),
            pltpu.SemaphoreType.DMA((NB // 2,)),
            pltpu.SemaphoreType.DMA((NB // 2,)),
            pltpu.SemaphoreType.DMA((NB // 2,)),
        ],
        compiler_params=pltpu.CompilerParams(collective_id=0),
    )(x)
